# baseline (speedup 1.0000x reference)
import functools

import jax
import jax.numpy as jnp
from jax import lax
from jax.experimental import pallas as pl
from jax.experimental.pallas import tpu as pltpu
from jax.experimental.pallas import tpu_sc as plsc

N = 1000
RPAD = 1024
B = 1024
SEQ = 20
L = 16
NB_GROUPS = 4
NN_GROUPS = 8
BG = B // NB_GROUPS
NR = 8
MAX_CHUNKS = 128 // NR


def _recip_body(x_ref, o_ref):
    x = x_ref[...]
    d = jnp.where(x == 0.0, jnp.float32(9999999.99), x)
    r = (1.0 / d).T
    o_ref[...] = jnp.concatenate(
        [r, jnp.full((N, RPAD - N), 1.0, jnp.float32)], axis=1)


_recip_call = pl.pallas_call(
    _recip_body,
    out_shape=jax.ShapeDtypeStruct((N, RPAD), jnp.float32),
)


_sc_mesh = plsc.VectorSubcoreMesh(core_axis_name="c", subcore_axis_name="s")


@functools.partial(
    pl.kernel,
    mesh=_sc_mesh,
    out_type=jax.ShapeDtypeStruct((SEQ, N, B), jnp.float32),
    compiler_params=pltpu.CompilerParams(
        use_tc_tiling_on_sc=True, needs_layout_passes=False),
    scratch_types=[
        pltpu.VMEM((N,), jnp.int32),
        pltpu.VMEM((BG * SEQ,), jnp.int32),
        pltpu.VMEM((SEQ, BG), jnp.int32),
        pltpu.VMEM((2, NR, RPAD), jnp.float32),
        pltpu.VMEM((2, SEQ, NR, BG), jnp.float32),
        pltpu.SemaphoreType.DMA,
        pltpu.SemaphoreType.DMA,
        pltpu.SemaphoreType.DMA,
        pltpu.SemaphoreType.DMA,
    ],
)
def _sc_gather(venue_hbm, poi_hbm, tablet_hbm, out_hbm,
               venue_v, poi_v, idx_v, rows_v, stg_v,
               sem_g0, sem_g1, sem_o0, sem_o1):
    wid = lax.axis_index("s") * 2 + lax.axis_index("c")
    bg = wid % NB_GROUPS
    ng = wid // NB_GROUPS
    b0 = bg * BG
    n0 = ng * 128
    n_chunks = jnp.where(ng == NN_GROUPS - 1,
                         (N - 128 * (NN_GROUPS - 1)) // NR, MAX_CHUNKS)
    sem_g = (sem_g0, sem_g1)
    sem_o = (sem_o0, sem_o1)

    pltpu.sync_copy(venue_hbm, venue_v)
    pltpu.sync_copy(poi_hbm.at[pl.ds(b0 * SEQ, BG * SEQ)], poi_v)

    def idx_body(s, carry):
        for k in range(BG // L):
            pos = (lax.iota(jnp.int32, L) + (k * L)) * SEQ + s
            v = plsc.load_gather(poi_v, [pos])
            idx_v[s, pl.ds(k * L, L)] = plsc.load_gather(venue_v, [v])
        return carry

    lax.fori_loop(0, SEQ, idx_body, 0)

    def start_gather(c, par):
        pltpu.async_copy(tablet_hbm.at[pl.ds(n0 + c * NR, NR)],
                         rows_v.at[par], sem_g[par])

    def wait_gather(par):
        pltpu.make_async_copy(tablet_hbm.at[pl.ds(n0, NR)],
                              rows_v.at[par], sem_g[par]).wait()

    def start_out(c, par):
        pltpu.async_copy(
            stg_v.at[par],
            out_hbm.at[:, pl.ds(n0 + c * NR, NR), pl.ds(b0, BG)],
            sem_o[par])

    def wait_out(par):
        pltpu.make_async_copy(
            stg_v.at[par],
            out_hbm.at[:, pl.ds(n0, NR), pl.ds(b0, BG)],
            sem_o[par]).wait()

    start_gather(0, 0)

    def chunk_work(c, par):
        @pl.when(c + 1 < n_chunks)
        def _():
            start_gather(c + 1, 1 - par)

        wait_gather(par)

        @pl.when(c >= 2)
        def _():
            wait_out(par)

        pz = jnp.full((L,), par, jnp.int32)

        def s_body(s, sc):
            for k in range(BG // L):
                iv = idx_v[s, pl.ds(k * L, L)]
                for nl in range(NR):
                    u = plsc.load_gather(
                        rows_v, [pz, jnp.full((L,), nl, jnp.int32), iv])
                    stg_v[par, s, nl, pl.ds(k * L, L)] = u
            return sc

        lax.fori_loop(0, SEQ, s_body, 0)
        start_out(c, par)

    def pair_body(i, carry):
        for par in (0, 1):
            c = 2 * i + par

            @pl.when(c < n_chunks)
            def _(c=c, par=par):
                chunk_work(c, par)
        return carry

    lax.fori_loop(0, MAX_CHUNKS // 2, pair_body, 0)
    wait_out(0)
    wait_out(1)


def kernel(venueid2coor, inputs_poi, poi_distance_matrix):
    tablet = _recip_call(poi_distance_matrix)
    out2 = _sc_gather(venueid2coor, inputs_poi.reshape(-1), tablet)
    return out2.transpose(2, 0, 1)


# speedup vs baseline: 1.5847x; 1.5847x over previous
import functools

import jax
import jax.numpy as jnp
from jax import lax
from jax.experimental import pallas as pl
from jax.experimental.pallas import tpu as pltpu
from jax.experimental.pallas import tpu_sc as plsc

N = 1000
RPAD = 1024
B = 1024
SEQ = 20
L = 16
NW = 32
BPW = 16
CH = 2 * SEQ


def _recip_body(x_ref, o_ref):
    x = x_ref[...]
    d = jnp.where(x == 0.0, jnp.float32(9999999.99), x)
    r = 1.0 / d
    o_ref[...] = jnp.concatenate(
        [r, jnp.full((N, RPAD - N), 1.0, jnp.float32)], axis=1)


_recip_call = pl.pallas_call(
    _recip_body,
    out_shape=jax.ShapeDtypeStruct((N, RPAD), jnp.float32),
)


_sc_mesh = plsc.VectorSubcoreMesh(core_axis_name="c", subcore_axis_name="s")


@functools.partial(
    pl.kernel,
    mesh=_sc_mesh,
    out_type=jax.ShapeDtypeStruct((B // 2, CH, RPAD), jnp.float32),
    compiler_params=pltpu.CompilerParams(
        use_tc_tiling_on_sc=True, needs_layout_passes=False),
    scratch_types=[
        pltpu.VMEM((N,), jnp.int32),
        pltpu.VMEM((2 * BPW * SEQ,), jnp.int32),
        pltpu.VMEM((BPW * CH,), jnp.int32),
        pltpu.VMEM((2, CH, RPAD), jnp.float32),
        pltpu.SemaphoreType.DMA,
        pltpu.SemaphoreType.DMA,
        pltpu.SemaphoreType.DMA,
        pltpu.SemaphoreType.DMA,
    ],
)
def _sc_gather(venue_hbm, poi_hbm, table_hbm, out_hbm,
               venue_v, poi_v, idx_v, rows_v,
               sem_g0, sem_g1, sem_o0, sem_o1):
    wid = lax.axis_index("s") * 2 + lax.axis_index("c")
    sem_g = (sem_g0, sem_g1)
    sem_o = (sem_o0, sem_o1)
    half = BPW * SEQ

    pltpu.sync_copy(venue_hbm, venue_v)
    pltpu.sync_copy(poi_hbm.at[pl.ds(wid * half, half)],
                    poi_v.at[pl.ds(0, half)])
    pltpu.sync_copy(poi_hbm.at[pl.ds((B // 2) * SEQ + wid * half, half)],
                    poi_v.at[pl.ds(half, half)])

    iota = lax.iota(jnp.int32, L)
    for c in range(BPW):
        base_a = c * SEQ
        base_b = half + c * SEQ
        p0 = iota + base_a
        p1 = jnp.where(iota + 16 < SEQ, iota + (base_a + 16),
                       iota + (base_b - 4))
        p2 = iota + (base_b + 4)
        for off, pos in ((0, p0), (16, p1), (24, p2)):
            v = plsc.load_gather(poi_v, [pos])
            idx_v[pl.ds(c * CH + off, L)] = plsc.load_gather(venue_v, [v])

    def start_gather(c):
        pltpu.async_copy(table_hbm.at[idx_v.at[pl.ds(c * CH, CH)]],
                         rows_v.at[c % 2], sem_g[c % 2])

    def wait_gather(c):
        pltpu.make_async_copy(table_hbm.at[idx_v.at[pl.ds(c * CH, CH)]],
                              rows_v.at[c % 2], sem_g[c % 2]).wait()

    def start_out(c):
        pltpu.async_copy(rows_v.at[c % 2], out_hbm.at[wid * BPW + c],
                         sem_o[c % 2])

    def wait_out(c):
        pltpu.make_async_copy(rows_v.at[c % 2], out_hbm.at[wid * BPW + c],
                              sem_o[c % 2]).wait()

    start_gather(0)
    for c in range(BPW):
        if c + 1 < BPW:
            start_gather(c + 1)
        wait_gather(c)
        if c >= 2:
            wait_out(c - 2)
        start_out(c)
    wait_out(BPW - 2)
    wait_out(BPW - 1)


def _tr_body(x_ref, o_ref):
    for s in range(SEQ):
        o_ref[s] = jnp.concatenate(
            [x_ref[:, s, :].T, x_ref[:, s + SEQ, :].T], axis=1)


_tr_call = pl.pallas_call(
    _tr_body,
    grid=(8,),
    in_specs=[pl.BlockSpec((B // 2, CH, 128), lambda i: (0, 0, i))],
    out_specs=pl.BlockSpec((SEQ, 128, B), lambda i: (0, i, 0)),
    out_shape=jax.ShapeDtypeStruct((SEQ, N, B), jnp.float32),
)


def kernel(venueid2coor, inputs_poi, poi_distance_matrix):
    table = _recip_call(poi_distance_matrix)
    r3 = _sc_gather(venueid2coor, inputs_poi.reshape(-1), table)
    out2 = _tr_call(r3)
    return out2.transpose(2, 0, 1)
